# baseline (speedup 1.0000x reference)
import functools

import jax
import jax.numpy as jnp
from jax import lax
from jax.experimental import pallas as pl
from jax.experimental.pallas import tpu as pltpu
from jax.experimental.pallas import tpu_sc as plsc

_N = 10000
_E = 40000
_IN = 128
_HID = 32
_OUT = 128

_EP = 40960
_EPH = _EP // 2
_NW = 32
_CHUNK = 128
_JA = 9
_JB = 1
_SCH = _EPH // (_NW * _CHUNK)
_PER_W = _EPH // _NW
_NP = 10112
_ROWS_W = _NP // 16
_MCOLS = 128

_BE = 2560
_BN = 400
_GCOLS = 33 * _HID


def _sc_gather(src_a, src_b, x):
  mesh = plsc.VectorSubcoreMesh(core_axis_name="c", subcore_axis_name="s")

  @functools.partial(
      pl.kernel,
      mesh=mesh,
      out_type=jax.ShapeDtypeStruct((_EPH, _IN), jnp.float32),
      scratch_types=[
          pltpu.VMEM((_JA, _CHUNK), jnp.int32),
          pltpu.VMEM((_JB, _CHUNK), jnp.int32),
          pltpu.VMEM((2, _CHUNK, _IN), jnp.float32),
          pltpu.SemaphoreType.DMA,
          pltpu.SemaphoreType.DMA,
      ],
  )
  def k(srca_hbm, srcb_hbm, x_hbm, out_hbm, idxa_v, idxb_v, rows_v,
        gsem, wsem):
    c = lax.axis_index("c")
    s = lax.axis_index("s")

    def run(idx_v, nch, base):
      gathers = [None] * nch
      writes = [None] * nch
      gathers[0] = pltpu.async_copy(x_hbm.at[idx_v.at[0]], rows_v.at[0], gsem)
      for j in range(nch):
        gathers[j].wait()
        if j >= 1:
          writes[j - 1].wait()
        if j + 1 < nch:
          gathers[j + 1] = pltpu.async_copy(
              x_hbm.at[idx_v.at[j + 1]], rows_v.at[(j + 1) % 2], gsem)
        writes[j] = pltpu.async_copy(
            rows_v.at[j % 2], out_hbm.at[pl.ds(base + j * _CHUNK, _CHUNK)],
            wsem)
      writes[nch - 1].wait()

    @pl.when(c == 0)
    def _():
      pltpu.sync_copy(srca_hbm.at[s], idxa_v)
      run(idxa_v, _JA, s * _JA * _CHUNK)

    @pl.when(c == 1)
    def _():
      pltpu.sync_copy(srcb_hbm.at[s], idxb_v)
      run(idxb_v, _JB, (16 * _JA + s * _JB) * _CHUNK)

  return k(src_a, src_b, x)


def _sc_scatter(dst_h, msg, init):
  mesh = plsc.VectorSubcoreMesh(core_axis_name="c", subcore_axis_name="s")

  @functools.partial(
      pl.kernel,
      mesh=mesh,
      out_type=jax.ShapeDtypeStruct((2, _NP, _MCOLS), jnp.float32),
      scratch_types=[
          pltpu.VMEM((_SCH, _CHUNK), jnp.int32),
          pltpu.VMEM((2, _CHUNK, _MCOLS), jnp.float32),
          pltpu.VMEM_SHARED((_NP, _MCOLS), jnp.float32),
          pltpu.SemaphoreType.DMA,
      ],
  )
  def k(dst_hbm, msg_hbm, init_hbm, out_hbm, idx_v, msg_v, agg_sh, lsem):
    c = lax.axis_index("c")
    s = lax.axis_index("s")
    wid = s * 2 + c
    loads = [None] * _SCH
    loads[0] = pltpu.async_copy(
        msg_hbm.at[pl.ds(wid * _PER_W, _CHUNK)], msg_v.at[0], lsem)
    pltpu.sync_copy(init_hbm.at[c].at[pl.ds(s * _ROWS_W, _ROWS_W)],
                    agg_sh.at[pl.ds(s * _ROWS_W, _ROWS_W)])
    pltpu.sync_copy(dst_hbm.at[wid], idx_v)
    plsc.subcore_barrier()
    for j in range(_SCH):
      loads[j].wait()
      if j + 1 < _SCH:
        loads[j + 1] = pltpu.async_copy(
            msg_hbm.at[pl.ds(wid * _PER_W + (j + 1) * _CHUNK, _CHUNK)],
            msg_v.at[(j + 1) % 2], lsem)
      pltpu.sync_copy(msg_v.at[j % 2], agg_sh.at[idx_v.at[j]], add=True)
    plsc.subcore_barrier()
    pltpu.sync_copy(agg_sh.at[pl.ds(s * _ROWS_W, _ROWS_W)],
                    out_hbm.at[c].at[pl.ds(s * _ROWS_W, _ROWS_W)])

  return k(dst_h, msg, init)


def _msg_body(xj_ref, ea_ref, w1_ref, b1_ref, w2_ref, r_ref, s_ref, out_ref):
  xj = xj_ref[...]
  ea = ea_ref[...]
  h = jnp.maximum(
      jnp.dot(ea, w1_ref[...], preferred_element_type=jnp.float32)
      + b1_ref[...], 0.0)
  g = jnp.dot(xj, w2_ref[...], preferred_element_type=jnp.float32)
  hb = jnp.dot(h, r_ref[...], preferred_element_type=jnp.float32)
  msg = jnp.dot(g * hb, s_ref[...], preferred_element_type=jnp.float32)
  cnt = (lax.broadcasted_iota(jnp.int32, (_BE, _MCOLS), 1) == _HID)
  out_ref[...] = msg + cnt.astype(jnp.float32)


def _tc_msg(x_j, ea_h, w1aug, b1aug, w2cat, rmat, smat):
  grid = _EPH // _BE
  return pl.pallas_call(
      _msg_body,
      grid=(grid,),
      in_specs=[
          pl.BlockSpec((_BE, _IN), lambda i: (i, 0)),
          pl.BlockSpec((_BE, 3), lambda i: (i, 0)),
          pl.BlockSpec((3, 33), lambda i: (0, 0)),
          pl.BlockSpec((1, 33), lambda i: (0, 0)),
          pl.BlockSpec((_IN, _GCOLS), lambda i: (0, 0)),
          pl.BlockSpec((33, _GCOLS), lambda i: (0, 0)),
          pl.BlockSpec((_GCOLS, _MCOLS), lambda i: (0, 0)),
      ],
      out_specs=pl.BlockSpec((_BE, _MCOLS), lambda i: (i, 0)),
      out_shape=jax.ShapeDtypeStruct((_EPH, _MCOLS), jnp.float32),
  )(x_j, ea_h, w1aug, b1aug, w2cat, rmat, smat)


def _final_body(x_ref, agg_ref, wr_ref, bc_ref, wfc_ref, bfc_ref, out_ref):
  x = x_ref[...]
  a = agg_ref[0] + agg_ref[1]
  mean = a[:, :_HID] / jnp.maximum(a[:, _HID:_HID + 1], 1.0)
  conv = (jnp.dot(x, wr_ref[...], preferred_element_type=jnp.float32)
          + mean + bc_ref[...])
  out_ref[...] = (jnp.dot(jnp.maximum(conv, 0.0), wfc_ref[...],
                          preferred_element_type=jnp.float32) + bfc_ref[...])


def _tc_final(x, agg, w_root, b_conv, wfc, bfc):
  grid = _N // _BN
  return pl.pallas_call(
      _final_body,
      grid=(grid,),
      in_specs=[
          pl.BlockSpec((_BN, _IN), lambda i: (i, 0)),
          pl.BlockSpec((2, _BN, _MCOLS), lambda i: (0, i, 0)),
          pl.BlockSpec((_IN, _HID), lambda i: (0, 0)),
          pl.BlockSpec((1, _HID), lambda i: (0, 0)),
          pl.BlockSpec((_HID, _OUT), lambda i: (0, 0)),
          pl.BlockSpec((1, _OUT), lambda i: (0, 0)),
      ],
      out_specs=pl.BlockSpec((_BN, _OUT), lambda i: (i, 0)),
      out_shape=jax.ShapeDtypeStruct((_N, _OUT), jnp.float32),
  )(x, agg, w_root, b_conv, wfc, bfc)


def kernel(x, edge_index, edge_attr, W1, b1, W2, b2, W_root, b_conv, Wfc, bfc):
  src = edge_index[0]
  dst = edge_index[1]
  src_p = jnp.pad(src, (0, _EP - _E))
  dst_p = jnp.pad(dst, (0, _EP - _E), constant_values=_N)
  def split_half(half):
    chunks = half.reshape(16 * (_JA + _JB), _CHUNK)
    return (chunks[:16 * _JA].reshape(16, _JA, _CHUNK),
            chunks[16 * _JA:].reshape(16, _JB, _CHUNK))

  src_a0, src_a1 = split_half(src_p[:_EPH])
  src_b0, src_b1 = split_half(src_p[_EPH:])
  dst_a = dst_p[:_EPH].reshape(_NW, _SCH, _CHUNK)
  dst_b = dst_p[_EPH:].reshape(_NW, _SCH, _CHUNK)
  ea_p = jnp.pad(edge_attr, ((0, _EP - _E), (0, 0)))
  w2cat = jnp.concatenate(
      [W2.reshape(_HID, _IN, _HID).transpose(1, 0, 2).reshape(_IN, _HID * _HID),
       b2.reshape(_IN, _HID)], axis=1)
  w1aug = jnp.concatenate([W1, jnp.zeros((3, 1), jnp.float32)], axis=1)
  b1aug = jnp.concatenate([b1, jnp.ones((1,), jnp.float32)]).reshape(1, 33)
  col = jnp.arange(_GCOLS)
  rmat = (col[None, :] // _HID == jnp.arange(33)[:, None]).astype(jnp.float32)
  smat = ((col % _HID)[:, None] == jnp.arange(_MCOLS)[None, :]
          ).astype(jnp.float32)
  zer = jnp.zeros((2, _NP, _MCOLS), jnp.float32)

  x_ja = _sc_gather(src_a0, src_a1, x)
  x_jb = _sc_gather(src_b0, src_b1, x)
  msg_a = _tc_msg(x_ja, ea_p[:_EPH], w1aug, b1aug, w2cat, rmat, smat)
  msg_b = _tc_msg(x_jb, ea_p[_EPH:], w1aug, b1aug, w2cat, rmat, smat)
  agg_a = _sc_scatter(dst_a, msg_a, zer)
  agg_b = _sc_scatter(dst_b, msg_b, agg_a)
  return _tc_final(x, agg_b, W_root, b_conv.reshape(1, _HID), Wfc,
                   bfc.reshape(1, _OUT))


# speedup vs baseline: 1.0313x; 1.0313x over previous
import functools

import jax
import jax.numpy as jnp
from jax import lax
from jax.experimental import pallas as pl
from jax.experimental.pallas import tpu as pltpu
from jax.experimental.pallas import tpu_sc as plsc

_N = 10000
_E = 40000
_IN = 128
_HID = 32
_OUT = 128

_EP = 40960
_EPH = _EP // 2
_NW = 32
_CHUNK = 128
_JA = 9
_JB = 1
_SCH = _EPH // (_NW * _CHUNK)
_PER_W = _EPH // _NW
_NP = 10112
_ROWS_W = _NP // 16
_MCOLS = 128

_BE = 2560
_BN = 400
_GCOLS = 33 * _HID


def _sc_gather(src_a, src_b, x):
  mesh = plsc.VectorSubcoreMesh(core_axis_name="c", subcore_axis_name="s")

  @functools.partial(
      pl.kernel,
      mesh=mesh,
      out_type=jax.ShapeDtypeStruct((_EPH, _IN), jnp.float32),
      scratch_types=[
          pltpu.VMEM((_JA, _CHUNK), jnp.int32),
          pltpu.VMEM((_JB, _CHUNK), jnp.int32),
          pltpu.VMEM((2, _CHUNK, _IN), jnp.float32),
          pltpu.SemaphoreType.DMA,
          pltpu.SemaphoreType.DMA,
      ],
  )
  def k(srca_hbm, srcb_hbm, x_hbm, out_hbm, idxa_v, idxb_v, rows_v,
        gsem, wsem):
    c = lax.axis_index("c")
    s = lax.axis_index("s")

    def run(idx_v, nch, base):
      gathers = [None] * nch
      writes = [None] * nch
      gathers[0] = pltpu.async_copy(x_hbm.at[idx_v.at[0]], rows_v.at[0], gsem)
      for j in range(nch):
        gathers[j].wait()
        if j >= 1:
          writes[j - 1].wait()
        if j + 1 < nch:
          gathers[j + 1] = pltpu.async_copy(
              x_hbm.at[idx_v.at[j + 1]], rows_v.at[(j + 1) % 2], gsem)
        writes[j] = pltpu.async_copy(
            rows_v.at[j % 2], out_hbm.at[pl.ds(base + j * _CHUNK, _CHUNK)],
            wsem)
      writes[nch - 1].wait()

    @pl.when(c == 0)
    def _():
      pltpu.sync_copy(srca_hbm.at[s], idxa_v)
      run(idxa_v, _JA, s * _JA * _CHUNK)

    @pl.when(c == 1)
    def _():
      pltpu.sync_copy(srcb_hbm.at[s], idxb_v)
      run(idxb_v, _JB, (16 * _JA + s * _JB) * _CHUNK)

  return k(src_a, src_b, x)


def _sc_scatter(dst_h, msg, zer):
  mesh = plsc.VectorSubcoreMesh(core_axis_name="c", subcore_axis_name="s")

  @functools.partial(
      pl.kernel,
      mesh=mesh,
      out_type=jax.ShapeDtypeStruct((2, _NP, _MCOLS), jnp.float32),
      scratch_types=[
          pltpu.VMEM((_SCH, _CHUNK), jnp.int32),
          pltpu.VMEM((2, _CHUNK, _MCOLS), jnp.float32),
          pltpu.VMEM_SHARED((_NP, _MCOLS), jnp.float32),
          pltpu.SemaphoreType.DMA,
      ],
  )
  def k(dst_hbm, msg_hbm, zer_hbm, out_hbm, idx_v, msg_v, agg_sh, lsem):
    c = lax.axis_index("c")
    s = lax.axis_index("s")
    wid = s * 2 + c
    loads = [None] * _SCH
    loads[0] = pltpu.async_copy(
        msg_hbm.at[pl.ds(wid * _PER_W, _CHUNK)], msg_v.at[0], lsem)
    pltpu.sync_copy(zer_hbm.at[pl.ds(s * _ROWS_W, _ROWS_W)],
                    agg_sh.at[pl.ds(s * _ROWS_W, _ROWS_W)])
    pltpu.sync_copy(dst_hbm.at[wid], idx_v)
    plsc.subcore_barrier()
    for j in range(_SCH):
      loads[j].wait()
      if j + 1 < _SCH:
        loads[j + 1] = pltpu.async_copy(
            msg_hbm.at[pl.ds(wid * _PER_W + (j + 1) * _CHUNK, _CHUNK)],
            msg_v.at[(j + 1) % 2], lsem)
      pltpu.sync_copy(msg_v.at[j % 2], agg_sh.at[idx_v.at[j]], add=True)
    plsc.subcore_barrier()
    pltpu.sync_copy(agg_sh.at[pl.ds(s * _ROWS_W, _ROWS_W)],
                    out_hbm.at[c].at[pl.ds(s * _ROWS_W, _ROWS_W)])

  return k(dst_h, msg, zer)


def _msg_body(xj_ref, ea_ref, w1_ref, b1_ref, w2_ref, r_ref, s_ref, out_ref):
  xj = xj_ref[...]
  ea = ea_ref[...]
  h = jnp.maximum(
      jnp.dot(ea, w1_ref[...], preferred_element_type=jnp.float32)
      + b1_ref[...], 0.0)
  g = jnp.dot(xj, w2_ref[...], preferred_element_type=jnp.float32)
  hb = jnp.dot(h, r_ref[...], preferred_element_type=jnp.float32)
  msg = jnp.dot(g * hb, s_ref[...], preferred_element_type=jnp.float32)
  cnt = (lax.broadcasted_iota(jnp.int32, (_BE, _MCOLS), 1) == _HID)
  out_ref[...] = msg + cnt.astype(jnp.float32)


def _tc_msg(x_j, ea_h, w1aug, b1aug, w2cat, rmat, smat):
  grid = _EPH // _BE
  return pl.pallas_call(
      _msg_body,
      grid=(grid,),
      in_specs=[
          pl.BlockSpec((_BE, _IN), lambda i: (i, 0)),
          pl.BlockSpec((_BE, 3), lambda i: (i, 0)),
          pl.BlockSpec((3, 33), lambda i: (0, 0)),
          pl.BlockSpec((1, 33), lambda i: (0, 0)),
          pl.BlockSpec((_IN, _GCOLS), lambda i: (0, 0)),
          pl.BlockSpec((33, _GCOLS), lambda i: (0, 0)),
          pl.BlockSpec((_GCOLS, _MCOLS), lambda i: (0, 0)),
      ],
      out_specs=pl.BlockSpec((_BE, _MCOLS), lambda i: (i, 0)),
      out_shape=jax.ShapeDtypeStruct((_EPH, _MCOLS), jnp.float32),
  )(x_j, ea_h, w1aug, b1aug, w2cat, rmat, smat)


def _final_body(x_ref, agga_ref, aggb_ref, wr_ref, bc_ref, wfc_ref, bfc_ref,
                out_ref):
  x = x_ref[...]
  a = (agga_ref[0] + agga_ref[1]) + (aggb_ref[0] + aggb_ref[1])
  mean = a[:, :_HID] / jnp.maximum(a[:, _HID:_HID + 1], 1.0)
  conv = (jnp.dot(x, wr_ref[...], preferred_element_type=jnp.float32)
          + mean + bc_ref[...])
  out_ref[...] = (jnp.dot(jnp.maximum(conv, 0.0), wfc_ref[...],
                          preferred_element_type=jnp.float32) + bfc_ref[...])


def _tc_final(x, agg_a, agg_b, w_root, b_conv, wfc, bfc):
  grid = _N // _BN
  agg_spec = pl.BlockSpec((2, _BN, _MCOLS), lambda i: (0, i, 0))
  return pl.pallas_call(
      _final_body,
      grid=(grid,),
      in_specs=[
          pl.BlockSpec((_BN, _IN), lambda i: (i, 0)),
          agg_spec,
          agg_spec,
          pl.BlockSpec((_IN, _HID), lambda i: (0, 0)),
          pl.BlockSpec((1, _HID), lambda i: (0, 0)),
          pl.BlockSpec((_HID, _OUT), lambda i: (0, 0)),
          pl.BlockSpec((1, _OUT), lambda i: (0, 0)),
      ],
      out_specs=pl.BlockSpec((_BN, _OUT), lambda i: (i, 0)),
      out_shape=jax.ShapeDtypeStruct((_N, _OUT), jnp.float32),
  )(x, agg_a, agg_b, w_root, b_conv, wfc, bfc)


def kernel(x, edge_index, edge_attr, W1, b1, W2, b2, W_root, b_conv, Wfc, bfc):
  src = edge_index[0]
  dst = edge_index[1]
  src_p = jnp.pad(src, (0, _EP - _E))
  dst_p = jnp.pad(dst, (0, _EP - _E), constant_values=_N)
  def split_half(half):
    chunks = half.reshape(16 * (_JA + _JB), _CHUNK)
    return (chunks[:16 * _JA].reshape(16, _JA, _CHUNK),
            chunks[16 * _JA:].reshape(16, _JB, _CHUNK))

  src_a0, src_a1 = split_half(src_p[:_EPH])
  src_b0, src_b1 = split_half(src_p[_EPH:])
  dst_a = dst_p[:_EPH].reshape(_NW, _SCH, _CHUNK)
  dst_b = dst_p[_EPH:].reshape(_NW, _SCH, _CHUNK)
  ea_p = jnp.pad(edge_attr, ((0, _EP - _E), (0, 0)))
  w2cat = jnp.concatenate(
      [W2.reshape(_HID, _IN, _HID).transpose(1, 0, 2).reshape(_IN, _HID * _HID),
       b2.reshape(_IN, _HID)], axis=1)
  w1aug = jnp.concatenate([W1, jnp.zeros((3, 1), jnp.float32)], axis=1)
  b1aug = jnp.concatenate([b1, jnp.ones((1,), jnp.float32)]).reshape(1, 33)
  col = jnp.arange(_GCOLS)
  rmat = (col[None, :] // _HID == jnp.arange(33)[:, None]).astype(jnp.float32)
  smat = ((col % _HID)[:, None] == jnp.arange(_MCOLS)[None, :]
          ).astype(jnp.float32)
  zer = jnp.zeros((_NP, _MCOLS), jnp.float32)

  x_ja = _sc_gather(src_a0, src_a1, x)
  x_jb = _sc_gather(src_b0, src_b1, x)
  msg_a = _tc_msg(x_ja, ea_p[:_EPH], w1aug, b1aug, w2cat, rmat, smat)
  msg_b = _tc_msg(x_jb, ea_p[_EPH:], w1aug, b1aug, w2cat, rmat, smat)
  agg_a = _sc_scatter(dst_a, msg_a, zer)
  agg_b = _sc_scatter(dst_b, msg_b, zer)
  return _tc_final(x, agg_a, agg_b, W_root, b_conv.reshape(1, _HID), Wfc,
                   bfc.reshape(1, _OUT))
